# Initial kernel scaffold; baseline (speedup 1.0000x reference)
import jax
import jax.numpy as jnp
from jax.experimental import pallas as pl


def kernel(x, edge_index, edge_attr, batch, Wn, bn, We, be, Wnbr, bnbr, a1, Wat1, g1_Wih, g1_Whh, g1_bih, g1_bhh, a2, Wat2, g2_Wih, g2_Whh, g2_bih, g2_bhh, am, Wam, gm_Wih, gm_Whh, gm_bih, gm_bhh, Wo, bo, gamma, beta):
    raise NotImplementedError("write your pallas kernel here")


# speedup vs baseline: 7.0749x; 7.0749x over previous
import functools

import jax
import jax.numpy as jnp
from jax import lax
from jax.experimental import pallas as pl
from jax.experimental.pallas import tpu as pltpu
from jax.experimental.pallas import tpu_sc as plsc

N = 10000
E = 320000
DIN = 128
DE = 16
H = 128
NH = 4
NG = 64

NC = 2
NS = 16
NW = NC * NS
L = 16
EPW = E // NW
CH = 80
NCHUNK = EPW // CH
NPAD = 10240
NPT = NPAD // NS

_f32 = jnp.float32


def _lrelu(x):
    return jnp.maximum(x, 0.01 * x)


def _elu(x):
    return jnp.where(x > 0, x, jnp.exp(x) - 1.0)


def _dot(a, b):
    return jnp.dot(a.astype(jnp.bfloat16), b.astype(jnp.bfloat16),
                   preferred_element_type=_f32)


def _tr32(x):
    return x.astype(jnp.bfloat16).astype(_f32)


def _dot2(a, b):
    ah = _tr32(a)
    return _dot(ah, b) + _dot(a - ah, b)


_CT = (((0,), (0,)), ((), ()))


def _dotgather(ohf, v):
    vh = _tr32(v)
    return _dot(ohf, vh) + _dot(ohf, v - vh)


def _dotseg(ohf, v):
    vh = _tr32(v)
    return (lax.dot_general(ohf, vh, _CT, preferred_element_type=_f32) +
            lax.dot_general(ohf, v - vh, _CT, preferred_element_type=_f32))


def _node_pre_body(x_ref, wn_ref, bn_ref, wt_ref, a1t_ref, h0_ref, hsp_ref, e1d_ref):
    h0 = _lrelu(_dot(x_ref[...], wn_ref[0]) + bn_ref[0])
    h0_ref[0] = h0
    hsp_ref[0] = _dot(h0, wt_ref[0])
    e1d_ref[0] = _dot(h0, a1t_ref[0])


def _node_pre(x, Wn, bn, WnbrT, a1t):
    return pl.pallas_call(
        _node_pre_body,
        grid=(NH,),
        in_specs=[
            pl.BlockSpec((N, DIN), lambda i: (0, 0)),
            pl.BlockSpec((1, DIN, H), lambda i: (i, 0, 0)),
            pl.BlockSpec((1, 1, H), lambda i: (i, 0, 0)),
            pl.BlockSpec((1, H, H), lambda i: (i, 0, 0)),
            pl.BlockSpec((1, H, 1), lambda i: (i, 0, 0)),
        ],
        out_specs=[
            pl.BlockSpec((1, N, H), lambda i: (i, 0, 0)),
            pl.BlockSpec((1, N, H), lambda i: (i, 0, 0)),
            pl.BlockSpec((1, N, 1), lambda i: (i, 0, 0)),
        ],
        out_shape=[
            jax.ShapeDtypeStruct((NH, N, H), _f32),
            jax.ShapeDtypeStruct((NH, N, H), _f32),
            jax.ShapeDtypeStruct((NH, N, 1), _f32),
        ],
    )(x, Wn, bn.reshape(NH, 1, H), WnbrT, a1t)


BE = 3200


def _edge_pre_body(ea_ref, we_ref, be_ref, wb_ref, bb_ref, out_ref):
    he = _lrelu(_dot(ea_ref[...], we_ref[0]) + be_ref[0])
    out_ref[0] = _dot(he, wb_ref[0]) + bb_ref[0]


def _edge_pre(ea, We, be, WnbrB, bnbr):
    return pl.pallas_call(
        _edge_pre_body,
        grid=(NH, E // BE),
        in_specs=[
            pl.BlockSpec((BE, DE), lambda i, j: (j, 0)),
            pl.BlockSpec((1, DE, H), lambda i, j: (i, 0, 0)),
            pl.BlockSpec((1, 1, H), lambda i, j: (i, 0, 0)),
            pl.BlockSpec((1, H, H), lambda i, j: (i, 0, 0)),
            pl.BlockSpec((1, 1, H), lambda i, j: (i, 0, 0)),
        ],
        out_specs=pl.BlockSpec((1, BE, H), lambda i, j: (i, j, 0)),
        out_shape=jax.ShapeDtypeStruct((NH, E, H), _f32),
    )(ea, We, be.reshape(NH, 1, H), WnbrB, bnbr.reshape(NH, 1, H))


def _gru_block(xin, h, wih, whh, bih, bhh):
    gi = _dot(xin, wih) + bih
    gh = _dot(h, whh) + bhh
    r = jax.nn.sigmoid(gi[:, :H] + gh[:, :H])
    z = jax.nn.sigmoid(gi[:, H:2 * H] + gh[:, H:2 * H])
    ng = jnp.tanh(gi[:, 2 * H:] + r * gh[:, 2 * H:])
    return (1.0 - z) * ng + z * h


NB = 2000


def _mid1_body(acc_ref, dp_ref, h0_ref, wat1_ref, wih_ref, whh_ref, bih_ref,
               bhh_ref, wat2_ref, a2t_ref, a2b_ref,
               h1_ref, t2v_ref, e2d_ref, e2s_ref):
    A = acc_ref[0] + acc_ref[1]
    s = jnp.sum(dp_ref[...], axis=1, keepdims=True)
    ctx1 = _elu(_dot2(A / (s + 1e-16), wat1_ref[...]))
    h1 = _gru_block(ctx1, h0_ref[0], wih_ref[...], whh_ref[...], bih_ref[...], bhh_ref[...])
    h1_ref[...] = h1
    t2v_ref[...] = _dot(h1, wat2_ref[...])
    e2d_ref[...] = _dot(h1, a2t_ref[...])
    e2s_ref[...] = _dot(h1, a2b_ref[...])


def _mid1(acc, dp3, h0_h, Wat1_h, wih, whh, bih, bhh, Wat2_h, a2t, a2b):
    return pl.pallas_call(
        _mid1_body,
        grid=(N // NB,),
        in_specs=[
            pl.BlockSpec((2, NB, H), lambda i: (0, i, 0)),
            pl.BlockSpec((NB, NW), lambda i: (i, 0)),
            pl.BlockSpec((1, NB, H), lambda i: (0, i, 0)),
            pl.BlockSpec((H, H), lambda i: (0, 0)),
            pl.BlockSpec((H, 3 * H), lambda i: (0, 0)),
            pl.BlockSpec((H, 3 * H), lambda i: (0, 0)),
            pl.BlockSpec((1, 3 * H), lambda i: (0, 0)),
            pl.BlockSpec((1, 3 * H), lambda i: (0, 0)),
            pl.BlockSpec((H, H), lambda i: (0, 0)),
            pl.BlockSpec((H, 1), lambda i: (0, 0)),
            pl.BlockSpec((H, 1), lambda i: (0, 0)),
        ],
        out_specs=[
            pl.BlockSpec((NB, H), lambda i: (i, 0)),
            pl.BlockSpec((NB, H), lambda i: (i, 0)),
            pl.BlockSpec((NB, 1), lambda i: (i, 0)),
            pl.BlockSpec((NB, 1), lambda i: (i, 0)),
        ],
        out_shape=[
            jax.ShapeDtypeStruct((N, H), _f32),
            jax.ShapeDtypeStruct((N, H), _f32),
            jax.ShapeDtypeStruct((N, 1), _f32),
            jax.ShapeDtypeStruct((N, 1), _f32),
        ],
    )(acc, dp3, h0_h.reshape(1, N, H), Wat1_h, wih, whh, bih.reshape(1, 3 * H),
      bhh.reshape(1, 3 * H), Wat2_h, a2t, a2b)


def _mid2_body(acc_ref, dp_ref, h1_ref, wih2_ref, whh2_ref, bih2_ref,
               bhh2_ref, h2_ref):
    A = acc_ref[0] + acc_ref[1]
    s = jnp.sum(dp_ref[...], axis=1, keepdims=True)
    ctx2 = _elu(A / (s + 1e-16))
    h2_ref[...] = _gru_block(ctx2, h1_ref[0], wih2_ref[...], whh2_ref[...],
                             bih2_ref[...], bhh2_ref[...])


def _mid2(acc, dp, h1_h, wih2, whh2, bih2, bhh2):
    return pl.pallas_call(
        _mid2_body,
        grid=(N // NB,),
        in_specs=[
            pl.BlockSpec((2, NB, H), lambda i: (0, i, 0)),
            pl.BlockSpec((NB, NW), lambda i: (i, 0)),
            pl.BlockSpec((1, NB, H), lambda i: (0, i, 0)),
            pl.BlockSpec((H, 3 * H), lambda i: (0, 0)),
            pl.BlockSpec((H, 3 * H), lambda i: (0, 0)),
            pl.BlockSpec((1, 3 * H), lambda i: (0, 0)),
            pl.BlockSpec((1, 3 * H), lambda i: (0, 0)),
        ],
        out_specs=pl.BlockSpec((NB, H), lambda i: (i, 0)),
        out_shape=jax.ShapeDtypeStruct((N, H), _f32),
    )(acc, dp, h1_h.reshape(1, N, H), wih2, whh2, bih2.reshape(1, 3 * H),
      bhh2.reshape(1, 3 * H))


def _readout_body(h2_ref, batch_ref, amt_ref, amb_ref, wam_ref,
                  wihm_ref, whhm_ref, bihm_ref, bhhm_ref, g_ref):
    h2 = h2_ref[...]
    ohb = batch_ref[...] == lax.broadcasted_iota(jnp.int32, (1, NG), 1)
    ohf = ohb.astype(_f32)
    g = _dotseg(ohf, h2)
    h2b = _tr32(h2)
    amt = amt_ref[...]
    amb = amb_ref[...]
    for _ in range(2):
        ga = _dot(g, amt)
        em = _lrelu(_dotgather(ohf, ga) + _dot(h2, amb))
        M = jnp.max(jnp.where(ohb, em, -jnp.inf), axis=0)
        M = jnp.where((M > -jnp.inf) & (M < jnp.inf), M, 0.0)
        exn = jnp.exp(em - _dot(ohf, M[:, None]))
        sg = _dotseg(ohf, exn)
        num = _dotseg(ohf, exn * h2b)
        ctxm = _elu(_dot2(num / (sg + 1e-16), wam_ref[...]))
        g = _gru_block(ctxm, g, wihm_ref[...], whhm_ref[...],
                       bihm_ref[...], bhhm_ref[...])
    g_ref[...] = g


def _readout(h2, batch2d, amt, amb, Wam_h, wihm, whhm, bihm, bhhm):
    return pl.pallas_call(
        _readout_body,
        out_shape=jax.ShapeDtypeStruct((NG, H), _f32),
    )(h2, batch2d, amt, amb, Wam_h, wihm, whhm,
      bihm.reshape(1, 3 * H), bhhm.reshape(1, 3 * H))


def _final_body(g_ref, wo_ref, bo_ref, gamma_ref, beta_ref, out_ref):
    y = bo_ref[...]
    for hd in range(NH):
        y = y + _dot(g_ref[hd], wo_ref[hd])
    mu = jnp.mean(y, axis=0, keepdims=True)
    var = jnp.mean((y - mu) ** 2, axis=0, keepdims=True)
    y = (y - mu) / jnp.sqrt(var + 1e-5)
    y = y * gamma_ref[...] + beta_ref[...]
    out_ref[...] = jnp.maximum(y, 0.0)


def _final(g_all, Wo_r, bo, gamma, beta):
    return pl.pallas_call(
        _final_body,
        out_shape=jax.ShapeDtypeStruct((NG, H), _f32),
    )(g_all, Wo_r, bo.reshape(1, H), gamma.reshape(1, H), beta.reshape(1, H))


@functools.lru_cache(maxsize=None)
def _mesh():
    return plsc.VectorSubcoreMesh(core_axis_name="c", subcore_axis_name="s",
                                  num_cores=NC, num_subcores=NS)


def _lane_mask(k):
    return lax.broadcasted_iota(jnp.int32, (L,), 0) == k


def _trunc_bf16(v):
    u = plsc.bitcast(v, jnp.uint32)
    u = u + jnp.uint32(0x7FFF) + ((u >> jnp.uint32(16)) & jnp.uint32(1))
    u = u & jnp.uint32(0xFFFF0000)
    return plsc.bitcast(u, _f32)


def _sc_scratch_l1():
    return [
        pltpu.VMEM((CH,), jnp.int32),
        pltpu.VMEM((CH,), jnp.int32),
        pltpu.VMEM((CH, H), _f32),
        pltpu.VMEM((CH, H), _f32),
        pltpu.VMEM((N,), _f32),
        pltpu.VMEM((H,), _f32),
        pltpu.VMEM((N,), _f32),
        pltpu.VMEM_SHARED((NPAD, H), _f32),
        pltpu.SemaphoreType.DMA,
    ]


def _sc_l1_body(src_h, dst_h, hsp_h, hep_h, e1d_h, a1b_h, zeros_h,
                outv_h, outd_h,
                srcv, dstv, rows, hep, e1dtab, a1btab, dpart, accS, sem):
    c = lax.axis_index("c")
    s = lax.axis_index("s")
    w = s * NC + c
    pltpu.sync_copy(e1d_h, e1dtab)
    pltpu.sync_copy(a1b_h, a1btab)
    pltpu.sync_copy(zeros_h.at[pl.ds(s * NPT, NPT)], accS.at[pl.ds(s * NPT, NPT)])

    def dz(i, carry):
        dpart[pl.ds(i * L, L)] = jnp.zeros((L,), _f32)
        return carry

    lax.fori_loop(0, N // L, dz, 0)
    plsc.subcore_barrier()

    a1bv = [a1btab[pl.ds(k * L, L)] for k in range(H // L)]

    def chunk(ci, carry):
        base = w * EPW + ci * CH
        pltpu.sync_copy(src_h.at[pl.ds(base, CH)], srcv)
        pltpu.sync_copy(dst_h.at[pl.ds(base, CH)], dstv)
        pltpu.async_copy(hsp_h.at[srcv], rows, sem).wait()
        pltpu.sync_copy(hep_h.at[pl.ds(base, CH)], hep)

        def group(g, carry2):
            dstv16 = dstv[pl.ds(g * L, L)]
            ed16 = plsc.load_gather(e1dtab, [dstv16])
            for k in range(L):
                i = g * L + k
                dacc = jnp.zeros((L,), _f32)
                nbrs = []
                for f in range(H // L):
                    t = rows[i, pl.ds(f * L, L)] + hep[i, pl.ds(f * L, L)]
                    nb = _trunc_bf16(jnp.maximum(t, 0.01 * t))
                    nbrs.append(nb)
                    dacc = dacc + nb * a1bv[f]
                e1 = ed16[k] + jnp.sum(dacc)
                e1 = jnp.maximum(e1, 0.01 * e1)
                exv = jnp.exp(jnp.full((L,), e1, _f32))
                for f in range(H // L):
                    hep[i, pl.ds(f * L, L)] = nbrs[f] * exv
                plsc.addupdate_scatter(dpart, [dstv16], exv, mask=_lane_mask(k))
            return carry2

        lax.fori_loop(0, CH // L, group, 0)
        pltpu.sync_copy(hep, accS.at[dstv], add=True)
        return carry

    lax.fori_loop(0, NCHUNK, chunk, 0)
    plsc.subcore_barrier()
    pltpu.sync_copy(accS.at[pl.ds(s * NPT, NPT)], outv_h.at[c, pl.ds(s * NPT, NPT)])
    pltpu.sync_copy(dpart, outd_h.at[w])


@functools.lru_cache(maxsize=None)
def _sc_l1():
    return pl.kernel(
        _sc_l1_body, mesh=_mesh(),
        out_type=(jax.ShapeDtypeStruct((NC, NPAD, H), _f32),
                  jax.ShapeDtypeStruct((NW, N), _f32)),
        scratch_types=_sc_scratch_l1(),
        compiler_params=pltpu.CompilerParams(needs_layout_passes=False),
    )


def _sc_scratch_l2():
    return [
        pltpu.VMEM((CH,), jnp.int32),
        pltpu.VMEM((CH,), jnp.int32),
        pltpu.VMEM((CH, H), _f32),
        pltpu.VMEM((N,), _f32),
        pltpu.VMEM((N,), _f32),
        pltpu.VMEM((N,), _f32),
        pltpu.VMEM_SHARED((NPAD, H), _f32),
        pltpu.SemaphoreType.DMA,
    ]


def _sc_l2_body(src_h, dst_h, t2v_h, e2d_h, e2s_h, zeros_h, outv_h, outd_h,
                srcv, dstv, rows, e2dtab, e2stab, dpart, accS, sem):
    c = lax.axis_index("c")
    s = lax.axis_index("s")
    w = s * NC + c
    pltpu.sync_copy(e2d_h, e2dtab)
    pltpu.sync_copy(e2s_h, e2stab)
    pltpu.sync_copy(zeros_h.at[pl.ds(s * NPT, NPT)], accS.at[pl.ds(s * NPT, NPT)])

    def dz(i, carry):
        dpart[pl.ds(i * L, L)] = jnp.zeros((L,), _f32)
        return carry

    lax.fori_loop(0, N // L, dz, 0)
    plsc.subcore_barrier()

    def chunk(ci, carry):
        base = w * EPW + ci * CH
        pltpu.sync_copy(src_h.at[pl.ds(base, CH)], srcv)
        pltpu.sync_copy(dst_h.at[pl.ds(base, CH)], dstv)
        pltpu.async_copy(t2v_h.at[srcv], rows, sem).wait()

        def group(g, carry2):
            dstv16 = dstv[pl.ds(g * L, L)]
            srcv16 = srcv[pl.ds(g * L, L)]
            ed16 = plsc.load_gather(e2dtab, [dstv16])
            es16 = plsc.load_gather(e2stab, [srcv16])
            e2v = ed16 + es16
            e2v = jnp.maximum(e2v, 0.01 * e2v)
            ex16 = jnp.exp(e2v)
            for k in range(L):
                i = g * L + k
                exv = jnp.full((L,), ex16[k], _f32)
                for f in range(H // L):
                    rows[i, pl.ds(f * L, L)] = rows[i, pl.ds(f * L, L)] * exv
                plsc.addupdate_scatter(dpart, [dstv16], ex16, mask=_lane_mask(k))
            return carry2

        lax.fori_loop(0, CH // L, group, 0)
        pltpu.sync_copy(rows, accS.at[dstv], add=True)
        return carry

    lax.fori_loop(0, NCHUNK, chunk, 0)
    plsc.subcore_barrier()
    pltpu.sync_copy(accS.at[pl.ds(s * NPT, NPT)], outv_h.at[c, pl.ds(s * NPT, NPT)])
    pltpu.sync_copy(dpart, outd_h.at[w])


@functools.lru_cache(maxsize=None)
def _sc_l2():
    return pl.kernel(
        _sc_l2_body, mesh=_mesh(),
        out_type=(jax.ShapeDtypeStruct((NC, NPAD, H), _f32),
                  jax.ShapeDtypeStruct((NW, N), _f32)),
        scratch_types=_sc_scratch_l2(),
        compiler_params=pltpu.CompilerParams(needs_layout_passes=False),
    )


def kernel(x, edge_index, edge_attr, batch, Wn, bn, We, be, Wnbr, bnbr, a1,
           Wat1, g1_Wih, g1_Whh, g1_bih, g1_bhh, a2, Wat2, g2_Wih, g2_Whh,
           g2_bih, g2_bhh, am, Wam, gm_Wih, gm_Whh, gm_bih, gm_bhh, Wo, bo,
           gamma, beta):
    src = edge_index[0]
    dst = edge_index[1]
    zacc = jnp.zeros((NPAD, H), _f32)
    batch2d = batch.reshape(N, 1)

    WnbrT = Wnbr[:, :H, :]
    WnbrB = Wnbr[:, H:, :]
    a1t = a1[:, :H, :]

    h0, hsP, e1d = _node_pre(x, Wn, bn, WnbrT, a1t)
    heP = _edge_pre(edge_attr, We, be, WnbrB, bnbr)

    def _tr(w):
        return w.astype(jnp.bfloat16).astype(_f32)

    gs = []
    for hd in range(NH):
        acc1, dp1 = _sc_l1()(src, dst, hsP[hd], heP[hd], e1d[hd, :, 0],
                             _tr(a1[hd, H:, 0]), zacc)
        h1, t2v, e2dv, e2sv = _mid1(acc1[:, :N], dp1.T, h0[hd],
                                    _tr(Wat1[hd]), g1_Wih[hd], g1_Whh[hd],
                                    g1_bih[hd], g1_bhh[hd], Wat2[hd],
                                    a2[hd, :H, :], a2[hd, H:, :])
        acc2, dp2 = _sc_l2()(src, dst, t2v, e2dv[:, 0], e2sv[:, 0], zacc)
        h2 = _mid2(acc2[:, :N], dp2.T, h1, g2_Wih[hd], g2_Whh[hd],
                   g2_bih[hd], g2_bhh[hd])
        g = _readout(h2, batch2d, am[hd, :H, :], am[hd, H:, :], _tr(Wam[hd]),
                     gm_Wih[hd], gm_Whh[hd], gm_bih[hd], gm_bhh[hd])
        gs.append(g)

    g_all = jnp.stack(gs, axis=0)
    return _final(g_all, Wo.reshape(NH, H, H), bo, gamma, beta)
